# baseline (speedup 1.0000x reference)
import functools

import jax
import jax.numpy as jnp
from jax import lax
from jax.experimental import pallas as pl
from jax.experimental.pallas import tpu as pltpu
from jax.experimental.pallas import tpu_sc as plsc

_NC = 2
_NS = 16
_NW = _NC * _NS

_LANES = 128
_SUBL = 8
_FRINGE_BLK = 64


def _table_body(embed_ref, lnw_ref, lnb_ref, wt_ref, bias_ref, main_ref, full16_ref):
    e = embed_ref[...]
    mean = jnp.mean(e, axis=1, keepdims=True)
    c = e - mean
    var = jnp.mean(c * c, axis=1, keepdims=True)
    h = (c / jnp.sqrt(var + 1e-5)) * lnw_ref[...][None, :] + lnb_ref[...][None, :]
    table = (
        jnp.dot(h, wt_ref[...], preferred_element_type=jnp.float32)
        + bias_ref[...][None, :]
    )
    split = main_ref.shape[1]
    main_ref[...] = table[:, :split]
    full16_ref[...] = table.astype(jnp.bfloat16)


def _compute_tables(embed, ln_w, ln_b, wt, bias, split):
    v = wt.shape[1]
    vocab = embed.shape[0]
    return pl.pallas_call(
        _table_body,
        out_shape=(
            jax.ShapeDtypeStruct((vocab, split), jnp.float32),
            jax.ShapeDtypeStruct((vocab, v), jnp.bfloat16),
        ),
    )(embed, ln_w, ln_b, wt, bias)


@functools.lru_cache(maxsize=None)
def _make_gather(bsz, seq, v, split, rows):
    b_per_w = bsz // _NW
    mesh = plsc.VectorSubcoreMesh(core_axis_name="c", subcore_axis_name="s")

    @functools.partial(
        pl.kernel,
        mesh=mesh,
        out_type=(),
        scratch_types=[
            pltpu.VMEM((seq,), jnp.int32),
            pltpu.VMEM((seq,), jnp.int32),
            pltpu.VMEM((rows, split), jnp.float32),
            pltpu.VMEM((rows, split), jnp.float32),
            pltpu.SemaphoreType.DMA,
            pltpu.SemaphoreType.DMA,
            pltpu.SemaphoreType.DMA,
            pltpu.SemaphoreType.DMA,
            pltpu.SemaphoreType.DMA,
            pltpu.SemaphoreType.DMA,
        ],
    )
    def gather(table_hbm, ids_hbm, out_hbm, idx0, idx1, rows0, rows1,
               i0, i1, g0, g1, w0, w1):
        wid = lax.axis_index("s") * _NC + lax.axis_index("c")
        base = wid * b_per_w
        idxs = (idx0, idx1)
        bufs = (rows0, rows1)
        isem = (i0, i1)
        gsem = (g0, g1)
        wsem = (w0, w1)

        def start_idx(k, s):
            pltpu.async_copy(ids_hbm.at[base + k], idxs[s], isem[s])

        def start_gather(s):
            pltpu.async_copy(table_hbm.at[idxs[s].at[pl.ds(0, rows)]], bufs[s], gsem[s])

        def start_write(k, s):
            pltpu.async_copy(
                bufs[s],
                out_hbm.at[base + k, pl.ds(0, rows), pl.ds(0, split)],
                wsem[s],
            )

        start_idx(0, 0)
        start_idx(1, 1)
        pltpu.make_async_copy(ids_hbm.at[base], idxs[0], isem[0]).wait()
        start_gather(0)

        def body(g, carry):
            for s in range(2):
                k = g * 2 + s
                cur, nxt = s, 1 - s

                @pl.when(k + 1 < b_per_w)
                def _():
                    pltpu.make_async_copy(ids_hbm.at[base], idxs[nxt], isem[nxt]).wait()

                pltpu.make_async_copy(
                    table_hbm.at[idxs[cur].at[pl.ds(0, rows)]], bufs[cur], gsem[cur]
                ).wait()

                @pl.when(k >= 1)
                def _():
                    pltpu.make_async_copy(
                        bufs[nxt],
                        out_hbm.at[base, pl.ds(0, rows), pl.ds(0, split)],
                        wsem[nxt],
                    ).wait()

                @pl.when(k + 1 < b_per_w)
                def _():
                    start_gather(nxt)

                start_write(k, cur)

                @pl.when(k + 2 < b_per_w)
                def _():
                    start_idx(k + 2, cur)
            return carry

        lax.fori_loop(0, b_per_w // 2, body, 0)
        pltpu.make_async_copy(
            bufs[1], out_hbm.at[base, pl.ds(0, rows), pl.ds(0, split)], wsem[1]
        ).wait()

    return gather


def _fringe_body(split, vocab, seq, rows, nb, idsf_ref, ids_ref, full16_ref,
                 out_ref, idsf_v, tails_ref, last_ref, sem_i, sem_t, sem_l):
    blk = pl.program_id(0)
    nlast = seq - rows
    half = nb // 2
    nbf = nb * seq
    iota = lax.broadcasted_iota(jnp.int32, (1, vocab), 1)
    cp_i = pltpu.make_async_copy(
        idsf_ref.at[pl.ds(blk * nbf, nbf)], idsf_v, sem_i
    )
    cp_i.start()
    full16 = full16_ref[...]
    tail16 = full16[:, split:]
    cp_i.wait()
    ids_all = idsf_v[...]
    for h in range(2):
        seg = ids_all[h * (nbf // 2):(h + 1) * (nbf // 2)]
        oh = (seg[:, None] == iota).astype(jnp.bfloat16)
        t_seg = jnp.dot(oh, tail16, preferred_element_type=jnp.float32)
        for j in range(half):
            tails_ref[h * half + j] = t_seg[j * seq:(j + 1) * seq, :]
    for l in range(nlast):
        ohl = (ids_ref[...][:, rows + l, None] == iota).astype(jnp.bfloat16)
        last_ref[:, l, :] = jnp.dot(ohl, full16, preferred_element_type=jnp.float32)
    cp_t = pltpu.make_async_copy(
        tails_ref.at[:, pl.ds(0, rows)],
        out_ref.at[pl.ds(blk * nb, nb), pl.ds(0, rows), pl.ds(split, vocab - split)],
        sem_t,
    )
    cp_t.start()
    cp_l = pltpu.make_async_copy(
        last_ref,
        out_ref.at[pl.ds(blk * nb, nb), pl.ds(rows, nlast), :],
        sem_l,
    )
    cp_l.start()
    cp_t.wait()
    cp_l.wait()


def _compute_fringe(ids, ids_flat, full16, bsz, seq, v, split, rows):
    vocab = full16.shape[0]
    nb = _FRINGE_BLK
    return pl.pallas_call(
        functools.partial(_fringe_body, split, vocab, seq, rows, nb),
        grid=(bsz // nb,),
        in_specs=[
            pl.BlockSpec(memory_space=pl.ANY),
            pl.BlockSpec((nb, seq), lambda i: (i, 0)),
            pl.BlockSpec((vocab, v), lambda i: (0, 0)),
        ],
        out_specs=pl.BlockSpec(memory_space=pl.ANY),
        out_shape=jax.ShapeDtypeStruct((bsz, seq, v), jnp.float32),
        scratch_shapes=[
            pltpu.VMEM((nb * seq,), jnp.int32),
            pltpu.VMEM((nb, seq, v - split), jnp.float32),
            pltpu.VMEM((nb, seq - rows, v), jnp.float32),
            pltpu.SemaphoreType.DMA,
            pltpu.SemaphoreType.DMA,
            pltpu.SemaphoreType.DMA,
        ],
    )(ids_flat, ids, full16)


def kernel(input_ids, embed, ln_w, ln_b, W, b):
    bsz, seq = input_ids.shape
    vocab = W.shape[0]
    split = (vocab // _LANES) * _LANES
    rows = (seq // _SUBL) * _SUBL
    main, full16 = _compute_tables(embed, ln_w, ln_b, W.T, b, split)
    ids = input_ids.astype(jnp.int32)
    out_fringe = _compute_fringe(ids, ids.reshape(-1), full16,
                                 bsz, seq, vocab, split, rows)
    out_ref = jax.new_ref(out_fringe)
    _make_gather(bsz, seq, vocab, split, rows)(main, ids, out_ref)
    return jax.ref.freeze(out_ref)


# speedup vs baseline: 1.3228x; 1.3228x over previous
import functools

import jax
import jax.numpy as jnp
from jax import lax
from jax.experimental import pallas as pl
from jax.experimental.pallas import tpu as pltpu
from jax.experimental.pallas import tpu_sc as plsc

_NC = 2
_NS = 16
_NW = _NC * _NS

_LANES = 128
_SUBL = 8


def _ln_body(embed_ref, lnw_ref, lnb_ref, out_ref):
    e = embed_ref[...]
    mean = jnp.mean(e, axis=1, keepdims=True)
    c = e - mean
    var = jnp.mean(c * c, axis=1, keepdims=True)
    out_ref[...] = (c / jnp.sqrt(var + 1e-5)) * lnw_ref[...][None, :] \
        + lnb_ref[...][None, :]


def _ln_table(embed, ln_w, ln_b):
    return pl.pallas_call(
        _ln_body,
        out_shape=jax.ShapeDtypeStruct(embed.shape, jnp.float32),
    )(embed, ln_w, ln_b)


@functools.lru_cache(maxsize=None)
def _make_hgather(n_tok, emb):
    per_w = n_tok // _NW
    chunk = 400
    n_chunks = per_w // chunk
    mesh = plsc.VectorSubcoreMesh(core_axis_name="c", subcore_axis_name="s")

    @functools.partial(
        pl.kernel,
        mesh=mesh,
        compiler_params=pltpu.CompilerParams(use_tc_tiling_on_sc=False),
        out_type=jax.ShapeDtypeStruct((n_tok, emb), jnp.float32),
        scratch_types=[
            pltpu.VMEM((per_w,), jnp.int32),
            pltpu.VMEM((chunk, emb), jnp.float32),
            pltpu.VMEM((chunk, emb), jnp.float32),
            pltpu.SemaphoreType.DMA,
            pltpu.SemaphoreType.DMA,
            pltpu.SemaphoreType.DMA,
            pltpu.SemaphoreType.DMA,
            pltpu.SemaphoreType.DMA,
        ],
    )
    def hgather(tab_hbm, ids_hbm, h_hbm, idx_v, buf0, buf1, isem, g0, g1, w0, w1):
        wid = lax.axis_index("s") * _NC + lax.axis_index("c")
        base = wid * per_w
        bufs = (buf0, buf1)
        gsem = (g0, g1)
        wsem = (w0, w1)
        pltpu.make_async_copy(ids_hbm.at[pl.ds(base, per_w)], idx_v, isem).start()
        pltpu.make_async_copy(ids_hbm.at[pl.ds(base, per_w)], idx_v, isem).wait()

        def start_gather(k, s):
            pltpu.async_copy(
                tab_hbm.at[idx_v.at[pl.ds(k * chunk, chunk)]], bufs[s], gsem[s]
            )

        def start_write(k, s):
            pltpu.async_copy(
                bufs[s], h_hbm.at[pl.ds(base + k * chunk, chunk)], wsem[s]
            )

        start_gather(0, 0)

        def body(g, carry):
            for s in range(2):
                k = g * 2 + s
                cur, nxt = s, 1 - s
                pltpu.make_async_copy(
                    tab_hbm.at[idx_v.at[pl.ds(0, chunk)]], bufs[cur], gsem[cur]
                ).wait()

                @pl.when(k >= 1)
                def _():
                    pltpu.make_async_copy(
                        bufs[nxt], h_hbm.at[pl.ds(base, chunk)], wsem[nxt]
                    ).wait()

                @pl.when(k + 1 < n_chunks)
                def _():
                    start_gather(k + 1, nxt)

                start_write(k, cur)
            return carry

        lax.fori_loop(0, n_chunks // 2, body, 0)
        pltpu.make_async_copy(
            bufs[1], h_hbm.at[pl.ds(base, chunk)], wsem[1]
        ).wait()

    return hgather


def _head_body(h_ref, w_ref, b_ref, out_ref):
    hblk = h_ref[...][0]
    w = w_ref[...]
    t = lax.dot_general(
        w, hblk, (((1,), (1,)), ((), ())),
        preferred_element_type=jnp.float32,
    )
    t = t + b_ref[...][:, None]
    vg = out_ref.shape[1]
    out_ref[...] = t.reshape(1, vg, 1, _SUBL, _LANES)


def _lm_head(h3, W, b, bsz, seq, vocab):
    vg = vocab // _SUBL
    bg = bsz // _LANES
    out5 = pl.pallas_call(
        _head_body,
        grid=(seq, bg),
        in_specs=[
            pl.BlockSpec((1, _LANES, h3.shape[2]), lambda l, g: (l, g, 0)),
            pl.BlockSpec((vocab, W.shape[1]), lambda l, g: (0, 0)),
            pl.BlockSpec((vocab,), lambda l, g: (0,)),
        ],
        out_specs=pl.BlockSpec((1, vg, 1, _SUBL, _LANES),
                               lambda l, g: (l, 0, g, 0, 0)),
        out_shape=jax.ShapeDtypeStruct((seq, vg, bg, _SUBL, _LANES), jnp.float32),
    )(h3, W, b)
    return out5


def kernel(input_ids, embed, ln_w, ln_b, W, b):
    bsz, seq = input_ids.shape
    vocab = W.shape[0]
    hn = _ln_table(embed, ln_w, ln_b)
    ids_t = input_ids.T.reshape(-1).astype(jnp.int32)
    h = _make_hgather(bsz * seq, embed.shape[1])(hn, ids_t)
    h3 = h.reshape(seq, bsz, embed.shape[1])
    out5 = _lm_head(h3, W, b, bsz, seq, vocab)
    out = jnp.transpose(out5, (2, 4, 0, 1, 3)).reshape(bsz, seq, vocab)
    return out


# speedup vs baseline: 1.3249x; 1.0015x over previous
import functools

import jax
import jax.numpy as jnp
from jax import lax
from jax.experimental import pallas as pl
from jax.experimental.pallas import tpu as pltpu
from jax.experimental.pallas import tpu_sc as plsc

_NC = 2
_NS = 16
_NW = _NC * _NS

_LANES = 128
_SUBL = 8


def _ln_body(embed_ref, lnw_ref, lnb_ref, out_ref):
    e = embed_ref[...]
    mean = jnp.mean(e, axis=1, keepdims=True)
    c = e - mean
    var = jnp.mean(c * c, axis=1, keepdims=True)
    out_ref[...] = (c / jnp.sqrt(var + 1e-5)) * lnw_ref[...][None, :] \
        + lnb_ref[...][None, :]


def _ln_table(embed, ln_w, ln_b):
    return pl.pallas_call(
        _ln_body,
        out_shape=jax.ShapeDtypeStruct(embed.shape, jnp.float32),
    )(embed, ln_w, ln_b)


@functools.lru_cache(maxsize=None)
def _make_hgather(n_tok, emb):
    per_w = n_tok // _NW
    chunk = 400
    n_chunks = per_w // chunk
    mesh = plsc.VectorSubcoreMesh(core_axis_name="c", subcore_axis_name="s")

    @functools.partial(
        pl.kernel,
        mesh=mesh,
        compiler_params=pltpu.CompilerParams(use_tc_tiling_on_sc=False),
        out_type=jax.ShapeDtypeStruct((n_tok, emb), jnp.float32),
        scratch_types=[
            pltpu.VMEM((per_w,), jnp.int32),
            pltpu.VMEM((chunk, emb), jnp.float32),
            pltpu.VMEM((chunk, emb), jnp.float32),
            pltpu.SemaphoreType.DMA,
            pltpu.SemaphoreType.DMA,
            pltpu.SemaphoreType.DMA,
            pltpu.SemaphoreType.DMA,
            pltpu.SemaphoreType.DMA,
        ],
    )
    def hgather(tab_hbm, ids_hbm, h_hbm, idx_v, buf0, buf1, isem, g0, g1, w0, w1):
        wid = lax.axis_index("s") * _NC + lax.axis_index("c")
        base = wid * per_w
        bufs = (buf0, buf1)
        gsem = (g0, g1)
        wsem = (w0, w1)
        pltpu.make_async_copy(ids_hbm.at[pl.ds(base, per_w)], idx_v, isem).start()
        pltpu.make_async_copy(ids_hbm.at[pl.ds(base, per_w)], idx_v, isem).wait()

        def start_gather(k, s):
            pltpu.async_copy(
                tab_hbm.at[idx_v.at[pl.ds(k * chunk, chunk)]], bufs[s], gsem[s]
            )

        def start_write(k, s):
            pltpu.async_copy(
                bufs[s], h_hbm.at[pl.ds(base + k * chunk, chunk)], wsem[s]
            )

        start_gather(0, 0)

        def body(g, carry):
            for s in range(2):
                k = g * 2 + s
                cur, nxt = s, 1 - s
                pltpu.make_async_copy(
                    tab_hbm.at[idx_v.at[pl.ds(0, chunk)]], bufs[cur], gsem[cur]
                ).wait()

                @pl.when(k >= 1)
                def _():
                    pltpu.make_async_copy(
                        bufs[nxt], h_hbm.at[pl.ds(base, chunk)], wsem[nxt]
                    ).wait()

                @pl.when(k + 1 < n_chunks)
                def _():
                    start_gather(k + 1, nxt)

                start_write(k, cur)
            return carry

        lax.fori_loop(0, n_chunks // 2, body, 0)
        pltpu.make_async_copy(
            bufs[1], h_hbm.at[pl.ds(base, chunk)], wsem[1]
        ).wait()

    return hgather


def _head_body(h_ref, w_ref, b_ref, out_ref):
    hblk = h_ref[...][0].astype(jnp.bfloat16)
    w = w_ref[...].astype(jnp.bfloat16)
    t = lax.dot_general(
        w, hblk, (((1,), (1,)), ((), ())),
        preferred_element_type=jnp.float32,
    )
    t = t + b_ref[...][:, None]
    vg = out_ref.shape[1]
    out_ref[...] = t.reshape(1, vg, 1, _SUBL, _LANES)


def _lm_head(h3, W, b, bsz, seq, vocab):
    vg = vocab // _SUBL
    bg = bsz // _LANES
    out5 = pl.pallas_call(
        _head_body,
        grid=(seq, bg),
        in_specs=[
            pl.BlockSpec((1, _LANES, h3.shape[2]), lambda l, g: (l, g, 0)),
            pl.BlockSpec((vocab, W.shape[1]), lambda l, g: (0, 0)),
            pl.BlockSpec((vocab,), lambda l, g: (0,)),
        ],
        out_specs=pl.BlockSpec((1, vg, 1, _SUBL, _LANES),
                               lambda l, g: (l, 0, g, 0, 0)),
        out_shape=jax.ShapeDtypeStruct((seq, vg, bg, _SUBL, _LANES), jnp.float32),
    )(h3, W, b)
    return out5


def kernel(input_ids, embed, ln_w, ln_b, W, b):
    bsz, seq = input_ids.shape
    vocab = W.shape[0]
    hn = _ln_table(embed, ln_w, ln_b)
    ids_t = input_ids.T.reshape(-1).astype(jnp.int32)
    h = _make_hgather(bsz * seq, embed.shape[1])(hn, ids_t)
    h3 = h.reshape(seq, bsz, embed.shape[1])
    out5 = _lm_head(h3, W, b, bsz, seq, vocab)
    out = jnp.transpose(out5, (2, 4, 0, 1, 3)).reshape(bsz, seq, vocab)
    return out
